# baseline (speedup 1.0000x reference)
import jax
import jax.numpy as jnp
from jax.experimental import pallas as pl
from jax.experimental.pallas import tpu as pltpu
from jax.experimental.pallas import tpu_sc as plsc

_NW = 32
_CHUNK = 8


def _router_body(x_ref, w_ref, logits_ref, idx_ref):
    xt = x_ref[...]
    lg = jax.lax.dot_general(
        xt, w_ref[...],
        dimension_numbers=(((1,), (1,)), ((), ())),
        preferred_element_type=jnp.float32,
        precision=jax.lax.Precision.DEFAULT,
    )
    logits_ref[...] = lg
    e = lg.shape[1]
    ids = jax.lax.broadcasted_iota(jnp.int32, lg.shape, 1)
    maxv = jnp.max(lg, axis=1, keepdims=True)
    idx_ref[...] = jnp.min(jnp.where(lg == maxv, ids, e), axis=1, keepdims=True)


def _sc_copy_body(x_hbm, out_hbm, buf):
    c = jax.lax.axis_index("c")
    s = jax.lax.axis_index("s")
    wid = s * 2 + c
    rows_per_w = x_hbm.shape[0] // _NW
    base = wid * rows_per_w
    nch = rows_per_w // _CHUNK

    def body(i, carry):
        r = base + i * _CHUNK
        pltpu.sync_copy(x_hbm.at[pl.ds(r, _CHUNK)], buf)
        pltpu.sync_copy(buf, out_hbm.at[pl.ds(r, _CHUNK)])
        return carry

    jax.lax.fori_loop(0, nch, body, 0)


def kernel(x, W):
    B, T, D = x.shape
    E = W.shape[0]
    N = B * T
    TILE = 512
    x2 = x.reshape(N, D)

    logits2, idx2 = pl.pallas_call(
        _router_body,
        grid=(N // TILE,),
        in_specs=[
            pl.BlockSpec((TILE, D), lambda i: (i, 0)),
            pl.BlockSpec((E, D), lambda i: (0, 0)),
        ],
        out_specs=[
            pl.BlockSpec((TILE, E), lambda i: (i, 0)),
            pl.BlockSpec((TILE, 1), lambda i: (i, 0)),
        ],
        out_shape=[
            jax.ShapeDtypeStruct((N, E), jnp.float32),
            jax.ShapeDtypeStruct((N, 1), jnp.int32),
        ],
    )(x2, W)

    out2 = pl.kernel(
        _sc_copy_body,
        out_type=jax.ShapeDtypeStruct((N, D), jnp.float32),
        mesh=plsc.VectorSubcoreMesh(core_axis_name="c", subcore_axis_name="s"),
        scratch_types=[pltpu.VMEM((_CHUNK, D), jnp.float32)],
    )(x2)

    output = out2.reshape(B, T, D)
    logits = logits2.reshape(B, T, E)
    indices = idx2.reshape(B, T, 1)
    weights = jnp.ones((B, T, 1), jnp.float32)
    return output, logits, indices, weights


# speedup vs baseline: 1.7137x; 1.7137x over previous
import jax
import jax.numpy as jnp
from jax.experimental import pallas as pl
from jax.experimental.pallas import tpu as pltpu


def _router_body(x_ref, w_ref, out_ref, logits_ref, idx_ref):
    xt = x_ref[...]
    out_ref[...] = xt
    lg = jax.lax.dot_general(
        xt, w_ref[...],
        dimension_numbers=(((1,), (1,)), ((), ())),
        preferred_element_type=jnp.float32,
        precision=jax.lax.Precision.DEFAULT,
    )
    logits_ref[...] = lg
    e = lg.shape[1]
    ids = jax.lax.broadcasted_iota(jnp.int32, lg.shape, 1)
    maxv = jnp.max(lg, axis=1, keepdims=True)
    idx_ref[...] = jnp.min(jnp.where(lg == maxv, ids, e), axis=1, keepdims=True)


def kernel(x, W):
    B, T, D = x.shape
    E = W.shape[0]
    N = B * T
    TILE = 512
    x2 = x.reshape(N, D)
    out2, logits2, idx2 = pl.pallas_call(
        _router_body,
        grid=(N // TILE,),
        compiler_params=pltpu.CompilerParams(
            dimension_semantics=("parallel",),
        ),
        in_specs=[
            pl.BlockSpec((TILE, D), lambda i: (i, 0)),
            pl.BlockSpec((E, D), lambda i: (0, 0)),
        ],
        out_specs=[
            pl.BlockSpec((TILE, D), lambda i: (i, 0)),
            pl.BlockSpec((TILE, E), lambda i: (i, 0)),
            pl.BlockSpec((TILE, 1), lambda i: (i, 0)),
        ],
        out_shape=[
            jax.ShapeDtypeStruct((N, D), jnp.float32),
            jax.ShapeDtypeStruct((N, E), jnp.float32),
            jax.ShapeDtypeStruct((N, 1), jnp.int32),
        ],
    )(x2, W)
    output = out2.reshape(B, T, D)
    logits = logits2.reshape(B, T, E)
    indices = idx2.reshape(B, T, 1)
    weights = jnp.ones((B, T, 1), jnp.float32)
    return output, logits, indices, weights
